# baseline (speedup 1.0000x reference)
import functools

import jax
import jax.numpy as jnp
from jax import lax
from jax.experimental import pallas as pl
from jax.experimental.pallas import tpu as pltpu
from jax.experimental.pallas import tpu_sc as plsc

NUM_EMB = 1000000
DIM = 16
BATCH = 16384
HIST = 20
NPOS = BATCH * HIST
NC = 2
NS = 16
FPC = DIM // NC
PPT = NPOS // NS
STRIPE = 62720
GCH = 4096
NGCH = PPT // GCH


def _tc_sample_kl(mu_t, ls_t, eps_t):
    bc = 16384
    grid = (NUM_EMB + bc - 1) // bc

    def body(mu_ref, ls_ref, eps_ref, w_ref, acc_ref):
        i = pl.program_id(0)
        m = mu_ref[...]
        l = ls_ref[...]
        e = eps_ref[...]
        sig = jnp.exp(l)
        w_ref[...] = m + sig * e
        col = i * bc + lax.broadcasted_iota(jnp.int32, (DIM, bc), 1)
        term = jnp.where(col < NUM_EMB,
                         sig * sig + m * m - 1.0 - 2.0 * l, 0.0)

        @pl.when(i == 0)
        def _():
            acc_ref[...] = jnp.zeros((1, 1), jnp.float32)

        acc_ref[...] += jnp.sum(term).reshape(1, 1)

    return pl.pallas_call(
        body,
        grid=(grid,),
        in_specs=[
            pl.BlockSpec((DIM, bc), lambda i: (0, i)),
            pl.BlockSpec((DIM, bc), lambda i: (0, i)),
            pl.BlockSpec((DIM, bc), lambda i: (0, i)),
        ],
        out_specs=[
            pl.BlockSpec((DIM, bc), lambda i: (0, i)),
            pl.BlockSpec((1, 1), lambda i: (0, 0)),
        ],
        out_shape=[
            jax.ShapeDtypeStruct((DIM, NUM_EMB), jnp.float32),
            jax.ShapeDtypeStruct((1, 1), jnp.float32),
        ],
    )(mu_t, ls_t, eps_t)


def _sc_gather_cols(w_t, idx_flat):
    mesh = plsc.VectorSubcoreMesh(
        core_axis_name="c", subcore_axis_name="s",
        num_cores=NC, num_subcores=NS)

    @functools.partial(
        pl.kernel,
        mesh=mesh,
        compiler_params=pltpu.CompilerParams(use_tc_tiling_on_sc=False),
        out_type=jax.ShapeDtypeStruct((DIM * NPOS,), jnp.float32),
        scratch_types=[
            pltpu.VMEM((PPT,), jnp.int32),
            pltpu.VMEM((GCH,), jnp.float32),
            pltpu.VMEM_SHARED((NUM_EMB,), jnp.float32),
            pltpu.SemaphoreType.DMA,
            pltpu.SemaphoreType.DMA,
        ],
    )
    def k(wt_hbm, idx_hbm, out_hbm, idx_v, gbuf, spline, sem_a, sem_g):
        c = lax.axis_index("c")
        t = lax.axis_index("s")
        pltpu.sync_copy(idx_hbm.at[pl.ds(t * PPT, PPT)], idx_v)

        def per_feature(kf, carry):
            f = c * FPC + kf
            base_a = jnp.where(t == NS - 1, NUM_EMB - STRIPE, t * STRIPE)
            pltpu.async_copy(
                wt_hbm.at[f, pl.ds(base_a, STRIPE)],
                spline.at[pl.ds(base_a, STRIPE)], sem_a).wait()
            plsc.subcore_barrier()

            def per_chunk(j, carry2):
                pltpu.async_copy(
                    spline.at[idx_v.at[pl.ds(j * GCH, GCH)]],
                    gbuf, sem_g).wait()
                pltpu.sync_copy(
                    gbuf,
                    out_hbm.at[pl.ds(f * NPOS + t * PPT + j * GCH, GCH)])
                return carry2

            lax.fori_loop(0, NGCH, per_chunk, 0)
            plsc.subcore_barrier()
            return carry

        lax.fori_loop(0, FPC, per_feature, 0)

    return k(w_t, idx_flat)


def kernel(input, mu, log_sigma, eps):
    w_t, kl_acc = _tc_sample_kl(mu.T, log_sigma.T, eps.T)
    idx_flat = input.T.reshape(-1)
    out_fm = _sc_gather_cols(w_t, idx_flat)
    embedding = out_fm.reshape(DIM, HIST, BATCH).transpose(2, 1, 0)
    kl = 0.5 * kl_acc[0, 0]
    return (embedding, kl)


# speedup vs baseline: 3.8868x; 3.8868x over previous
import functools

import jax
import jax.numpy as jnp
from jax import lax
from jax.experimental import pallas as pl
from jax.experimental.pallas import tpu as pltpu
from jax.experimental.pallas import tpu_sc as plsc

NUM_EMB = 1000000
DIM = 16
BATCH = 16384
HIST = 20
NPOS = BATCH * HIST
NTC = (NUM_EMB + 127) // 128
NC = 2
NS = 16
FPC = DIM // NC
PPT = NPOS // NS
GCH = 4096
NGCH = PPT // GCH


def _tc_sample_kl(mu_t, ls_t, eps_t):
    bc = 16384
    grid = (NUM_EMB + bc - 1) // bc

    def body(mu_ref, ls_ref, eps_ref, w_ref, acc_ref):
        i = pl.program_id(0)
        m = mu_ref[...]
        l = ls_ref[...]
        e = eps_ref[...]
        sig = jnp.exp(l)
        wt = m + sig * e
        w_ref[...] = wt.reshape(2, 8, bc // 128, 128).transpose(0, 2, 1, 3)
        col = i * bc + lax.broadcasted_iota(jnp.int32, (DIM, bc), 1)
        term = jnp.where(col < NUM_EMB,
                         sig * sig + m * m - 1.0 - 2.0 * l, 0.0)

        @pl.when(i == 0)
        def _():
            acc_ref[...] = jnp.zeros((1, 1), jnp.float32)

        acc_ref[...] += jnp.sum(term).reshape(1, 1)

    return pl.pallas_call(
        body,
        grid=(grid,),
        in_specs=[
            pl.BlockSpec((DIM, bc), lambda i: (0, i)),
            pl.BlockSpec((DIM, bc), lambda i: (0, i)),
            pl.BlockSpec((DIM, bc), lambda i: (0, i)),
        ],
        out_specs=[
            pl.BlockSpec((2, bc // 128, 8, 128), lambda i: (0, i, 0, 0)),
            pl.BlockSpec((1, 1), lambda i: (0, 0)),
        ],
        out_shape=[
            jax.ShapeDtypeStruct((2, NTC, 8, 128), jnp.float32),
            jax.ShapeDtypeStruct((1, 1), jnp.float32),
        ],
    )(mu_t, ls_t, eps_t)


def _sc_gather(w_flat, idx_flat):
    mesh = plsc.VectorSubcoreMesh(
        core_axis_name="c", subcore_axis_name="s",
        num_cores=NC, num_subcores=NS)

    @functools.partial(
        pl.kernel,
        mesh=mesh,
        compiler_params=pltpu.CompilerParams(use_tc_tiling_on_sc=False),
        out_type=jax.ShapeDtypeStruct((DIM * NPOS,), jnp.float32),
        scratch_types=[
            pltpu.VMEM((PPT,), jnp.int32),
            pltpu.VMEM((PPT,), jnp.int32),
            pltpu.VMEM((GCH,), jnp.float32),
            pltpu.VMEM((GCH,), jnp.float32),
            pltpu.SemaphoreType.DMA,
            pltpu.SemaphoreType.DMA,
        ],
    )
    def k(w_hbm, idx_hbm, out_hbm, idx_v, idx1_v, gb0, gb1, sem0, sem1):
        c = lax.axis_index("c")
        t = lax.axis_index("s")
        pltpu.sync_copy(idx_hbm.at[pl.ds(t * PPT, PPT)], idx_v)
        nv = PPT // DIM

        def xform(r, carry):
            iv = idx_v[pl.ds(r * DIM, DIM)]
            idx_v[pl.ds(r * DIM, DIM)] = (iv >> 7) * 1024 + (iv & 127)
            return carry

        lax.fori_loop(0, nv, xform, 0, unroll=8)
        bufs = (gb0, gb1)
        sems = (sem0, sem1)

        for kf in range(FPC):
            off_f = c * (NTC * 1024) + kf * 128

            def shift(r, carry, _off=off_f, _kf=kf):
                iv = idx_v[pl.ds(r * DIM, DIM)]
                if _kf == 0:
                    idx1_v[pl.ds(r * DIM, DIM)] = iv + _off
                else:
                    idx1_v[pl.ds(r * DIM, DIM)] = (
                        idx1_v[pl.ds(r * DIM, DIM)] + 128)
                return carry

            lax.fori_loop(0, nv, shift, 0, unroll=8)
            obase = (c * FPC + kf) * NPOS + t * PPT
            cps = [None, None]
            for j in range(NGCH + 1):
                if j < NGCH:
                    cps[j % 2] = pltpu.async_copy(
                        w_hbm.at[idx1_v.at[pl.ds(j * GCH, GCH)]],
                        bufs[j % 2], sems[j % 2])
                if j > 0:
                    cps[(j - 1) % 2].wait()
                    pltpu.sync_copy(
                        bufs[(j - 1) % 2],
                        out_hbm.at[pl.ds(obase + (j - 1) * GCH, GCH)])

    return k(w_flat, idx_flat)


def kernel(input, mu, log_sigma, eps):
    w4, kl_acc = _tc_sample_kl(mu.T, log_sigma.T, eps.T)
    idx_flat = input.T.reshape(-1)
    out_fm = _sc_gather(w4.reshape(-1), idx_flat)
    embedding = out_fm.reshape(DIM, HIST, BATCH).transpose(2, 1, 0)
    kl = 0.5 * kl_acc[0, 0]
    return (embedding, kl)
